# baseline (speedup 1.0000x reference)
import functools

import jax
import jax.numpy as jnp
from jax import lax
from jax.experimental import pallas as pl
from jax.experimental.pallas import tpu as pltpu
from jax.experimental.pallas import tpu_sc as plsc

N = 10000
E = 320000
IN_DIM = 128
HID = 16
H = 8
NEG = 0.2

NT = 16
TE = E // NT
CB = 80
NCH = TE // CB
RB = 624
RW = 48


def _leaky(x):
    return jnp.where(x > 0, x, NEG * x)


def _vgather(v, idx):
    return lax.gather(
        v, idx[:, None],
        dimension_numbers=lax.GatherDimensionNumbers(
            offset_dims=(), collapsed_slice_dims=(0,), start_index_map=(0,)),
        slice_sizes=(1,), mode=lax.GatherScatterMode.PROMISE_IN_BOUNDS)


def _pre_body(x0_ref, x1_ref, Wn_ref, rW_ref, rb_ref, em0_ref, em1_ref,
              Ws0_ref, Wd0_ref, Ws1_ref, Wd1_ref, pW0_ref, pb0_ref,
              pW1_ref, pb1_ref, G_ref,
              f0_ref, f1_ref, ts0_ref, td0_ref, ts1_ref, td1_ref,
              r0_ref, r1_ref, p0_ref, p1_ref):
    Wn = Wn_ref[...]
    G = G_ref[...]
    x0 = x0_ref[...]
    x1 = x1_ref[...]
    f0 = jnp.dot(x0, Wn, preferred_element_type=jnp.float32)
    f1 = jnp.dot(x1, Wn, preferred_element_type=jnp.float32)
    f0_ref[...] = f0
    f1_ref[...] = f1
    rv_s0 = jnp.dot(em0_ref[...], Ws0_ref[...], preferred_element_type=jnp.float32)
    rv_d0 = jnp.dot(em0_ref[...], Wd0_ref[...], preferred_element_type=jnp.float32)
    rv_s1 = jnp.dot(em1_ref[...], Ws1_ref[...], preferred_element_type=jnp.float32)
    rv_d1 = jnp.dot(em1_ref[...], Wd1_ref[...], preferred_element_type=jnp.float32)
    ts0_ref[...] = jnp.dot(f0 * rv_s0, G, preferred_element_type=jnp.float32)
    td0_ref[...] = jnp.dot(f0 * rv_d0, G, preferred_element_type=jnp.float32)
    ts1_ref[...] = jnp.dot(f1 * rv_s1, G, preferred_element_type=jnp.float32)
    td1_ref[...] = jnp.dot(f1 * rv_d1, G, preferred_element_type=jnp.float32)
    r0_ref[...] = jnp.dot(x0, rW_ref[...], preferred_element_type=jnp.float32) + rb_ref[...]
    r1_ref[...] = jnp.dot(x1, rW_ref[...], preferred_element_type=jnp.float32) + rb_ref[...]
    p0_ref[...] = jnp.dot(em0_ref[...], pW0_ref[...], preferred_element_type=jnp.float32) + pb0_ref[...]
    p1_ref[...] = jnp.dot(em1_ref[...], pW1_ref[...], preferred_element_type=jnp.float32) + pb1_ref[...]


def _run_pre(x0, x1, Wn, rW, rb, em0, em1, Ws0, Wd0, Ws1, Wd1,
             pW0, pb0, pW1, pb1, G):
    BR = 1000
    grid = (N // BR,)
    row = lambda i: (i, 0)
    full = lambda i: (0, 0)
    blk = lambda shp, m: pl.BlockSpec(shp, m)
    out_shapes = (
        jax.ShapeDtypeStruct((N, 128), jnp.float32),
        jax.ShapeDtypeStruct((N, 128), jnp.float32),
        jax.ShapeDtypeStruct((N, 16), jnp.float32),
        jax.ShapeDtypeStruct((N, 16), jnp.float32),
        jax.ShapeDtypeStruct((N, 16), jnp.float32),
        jax.ShapeDtypeStruct((N, 16), jnp.float32),
        jax.ShapeDtypeStruct((N, 128), jnp.float32),
        jax.ShapeDtypeStruct((N, 128), jnp.float32),
        jax.ShapeDtypeStruct((1, 512), jnp.float32),
        jax.ShapeDtypeStruct((1, 512), jnp.float32),
    )
    in_specs = [
        blk((BR, 128), row), blk((BR, 128), row),
        blk((128, 128), full), blk((128, 128), full), blk((1, 128), full),
        blk((1, 64), full), blk((1, 64), full),
        blk((64, 128), full), blk((64, 128), full),
        blk((64, 128), full), blk((64, 128), full),
        blk((64, 512), full), blk((1, 512), full),
        blk((64, 512), full), blk((1, 512), full),
        blk((128, 16), full),
    ]
    out_specs = (
        blk((BR, 128), row), blk((BR, 128), row),
        blk((BR, 16), row), blk((BR, 16), row),
        blk((BR, 16), row), blk((BR, 16), row),
        blk((BR, 128), row), blk((BR, 128), row),
        blk((1, 512), full), blk((1, 512), full),
    )
    return pl.pallas_call(
        _pre_body, grid=grid, in_specs=in_specs, out_specs=out_specs,
        out_shape=out_shapes,
    )(x0, x1, Wn, rW, rb, em0, em1, Ws0, Wd0, Ws1, Wd1, pW0, pb0, pW1, pb1, G)


CB2 = 96
NCH2 = 210
TEP = NCH2 * CB2
NP = N + 16
NB2 = NCH2 // 2


def _sc_body(pidx_hbm, tsrc_hbm, tdst_hbm, feat_hbm, out_hbm,
             pidx2, vdst0, vdst1, voffs0, voffs1, voffd0, voffd1,
             gs0, gs1, gd0, gd1, wv0, wv1, rr0, rr1, fb0, fb1,
             ssum_sh, out_sh,
             sa0, sb0, sc0, sd0, sa1, sb1, sc1, sd1):
    c = lax.axis_index("c")
    s = lax.axis_index("s")
    cN = c * N
    erow = (c * NT + s) * NCH2
    rbase = s * RB
    zero16 = jnp.zeros((16,), jnp.float32)
    hsplat = [jnp.full((16,), h, jnp.int32) for h in range(H)]
    vdst = (vdst0, vdst1)
    voffs = (voffs0, voffs1)
    voffd = (voffd0, voffd1)
    gs = (gs0, gs1)
    gd = (gd0, gd1)
    wv = (wv0, wv1)
    rr = (rr0, rr1)
    fb = (fb0, fb1)
    sems = ((sa0, sb0, sc0, sd0), (sa1, sb1, sc1, sd1))

    def load_pair(j0):
        pltpu.sync_copy(pidx_hbm.at[pl.ds(erow + j0, 2)], pidx2)

    def unpack(b):
        for i in range(CB2 // 16):
            p = pidx2[b, pl.ds(i * 16, 16)]
            d = p >> 16
            sr = p & 0xFFFF
            vdst[b][pl.ds(i * 16, 16)] = d
            voffd[b][pl.ds(i * 16, 16)] = d + cN
            voffs[b][pl.ds(i * 16, 16)] = sr + cN

    def zf(i, _):
        fb0[i // 8, pl.ds((i % 8) * 16, 16)] = zero16
        return 0
    lax.fori_loop(0, CB2 * 8, zf, 0)

    def zw(i, _):
        wv0[i, :] = zero16
        return 0
    lax.fori_loop(0, CB2, zw, 0)

    def zrows(j, _):
        r = rbase + j * RW
        pltpu.sync_copy(wv0.at[pl.ds(0, RW)], ssum_sh.at[pl.ds(r, RW)])
        pltpu.sync_copy(fb0.at[pl.ds(0, RW)], out_sh.at[pl.ds(r, RW)])
        return 0
    lax.fori_loop(0, RB // RW, zrows, 0)

    @pl.when(s == NT - 1)
    def _():
        pltpu.sync_copy(wv0.at[pl.ds(0, 32)], ssum_sh.at[pl.ds(NT * RB, 32)])
        pltpu.sync_copy(fb0.at[pl.ds(0, 32)], out_sh.at[pl.ds(NT * RB, 32)])

    plsc.subcore_barrier()

    def a_compute_scatter(b):
        for e in range(CB2):
            v = gs[b][e, :] + gd[b][e, :]
            wv[b][e, :] = jnp.exp(jnp.where(v > 0, v, NEG * v))
        pltpu.sync_copy(wv[b], ssum_sh.at[vdst[b]], add=True)

    def passA(m, _):
        j0 = 2 * m
        load_pair(j0)
        unpack(0)
        d0 = (pltpu.async_copy(tsrc_hbm.at[voffs[0]], gs[0], sems[0][0]),
              pltpu.async_copy(tdst_hbm.at[voffd[0]], gd[0], sems[0][1]))
        unpack(1)
        d1 = (pltpu.async_copy(tsrc_hbm.at[voffs[1]], gs[1], sems[1][0]),
              pltpu.async_copy(tdst_hbm.at[voffd[1]], gd[1], sems[1][1]))
        d0[0].wait()
        d0[1].wait()
        a_compute_scatter(0)
        d1[0].wait()
        d1[1].wait()
        a_compute_scatter(1)
        return 0
    lax.fori_loop(0, NB2, passA, 0)

    plsc.subcore_barrier()

    def recip(j, _):
        r = rbase + j * RW
        pltpu.sync_copy(ssum_sh.at[pl.ds(r, RW)], wv0.at[pl.ds(0, RW)])
        for i in range(RW):
            wv0[i, :] = 1.0 / (wv0[i, :] + 1e-16)
        pltpu.sync_copy(wv0.at[pl.ds(0, RW)], ssum_sh.at[pl.ds(r, RW)])
        return 0
    lax.fori_loop(0, RB // RW, recip, 0)

    @pl.when(s == NT - 1)
    def _():
        pltpu.sync_copy(ssum_sh.at[pl.ds(NT * RB, 16)], wv0.at[pl.ds(0, 16)])
        for i in range(16):
            wv0[i, :] = 1.0 / (wv0[i, :] + 1e-16)
        pltpu.sync_copy(wv0.at[pl.ds(0, 16)], ssum_sh.at[pl.ds(NT * RB, 16)])

    plsc.subcore_barrier()

    def b_issue(b):
        return (pltpu.async_copy(feat_hbm.at[voffs[b]], fb[b], sems[b][0]),
                pltpu.async_copy(tsrc_hbm.at[voffs[b]], gs[b], sems[b][1]),
                pltpu.async_copy(tdst_hbm.at[voffd[b]], gd[b], sems[b][2]),
                pltpu.async_copy(ssum_sh.at[vdst[b]], rr[b], sems[b][3]))

    def b_compute_scatter(b):
        def scale(p, _):
            for q in range(2):
                e = 2 * p + q
                v = gs[b][e, :] + gd[b][e, :]
                w = jnp.exp(jnp.where(v > 0, v, NEG * v))
                av = w * rr[b][e, :]
                for h in range(H):
                    bv = _vgather(av, hsplat[h])
                    fb[b][e, pl.ds(h * 16, 16)] = fb[b][e, pl.ds(h * 16, 16)] * bv
            return 0
        lax.fori_loop(0, CB2 // 2, scale, 0)
        pltpu.sync_copy(fb[b], out_sh.at[vdst[b]], add=True)

    def passB(m, _):
        j0 = 2 * m
        load_pair(j0)
        unpack(0)
        d0 = b_issue(0)
        unpack(1)
        d1 = b_issue(1)
        for d in d0:
            d.wait()
        b_compute_scatter(0)
        for d in d1:
            d.wait()
        b_compute_scatter(1)
        return 0
    lax.fori_loop(0, NB2, passB, 0)

    plsc.subcore_barrier()

    def wout(j, _):
        r = rbase + j * RW
        pltpu.sync_copy(out_sh.at[pl.ds(r, RW)], fb0.at[pl.ds(0, RW)])
        pltpu.sync_copy(fb0.at[pl.ds(0, RW)], out_hbm.at[pl.ds(cN + r, RW)])
        return 0
    lax.fori_loop(0, RB // RW, wout, 0)

    @pl.when(s == NT - 1)
    def _():
        pltpu.sync_copy(out_sh.at[pl.ds(NT * RB, 16)], fb0.at[pl.ds(0, 16)])
        pltpu.sync_copy(fb0.at[pl.ds(0, 16)], out_hbm.at[pl.ds(cN + NT * RB, 16)])


def _run_sc(pidx, tsrc_all, tdst_all, feat_all):
    mesh = plsc.VectorSubcoreMesh(core_axis_name="c", subcore_axis_name="s",
                                  num_cores=2, num_subcores=NT)
    f = pl.kernel(
        _sc_body,
        out_type=jax.ShapeDtypeStruct((2 * N, 128), jnp.float32),
        mesh=mesh,
        compiler_params=pltpu.CompilerParams(use_tc_tiling_on_sc=False),
        scratch_types=(
            [pltpu.VMEM((2, CB2), jnp.int32)]
            + [pltpu.VMEM((CB2,), jnp.int32)] * 6
            + [pltpu.VMEM((CB2, 16), jnp.float32)] * 8
            + [pltpu.VMEM((CB2, 128), jnp.float32)] * 2
            + [pltpu.VMEM_SHARED((NP, 16), jnp.float32),
               pltpu.VMEM_SHARED((NP, 128), jnp.float32)]
            + [pltpu.SemaphoreType.DMA] * 8),
    )
    return f(pidx, tsrc_all, tdst_all, feat_all)


def _post_body(s0_ref, s1_ref, r0_ref, r1_ref, ab_ref, a0_ref, a1_ref, G_ref,
               E8_ref, c0_ref, c1_ref):
    ab = ab_ref[...]
    G = G_ref[...]
    E8 = E8_ref[...]
    o0 = jnp.maximum(s0_ref[...], 0.0) * ab + r0_ref[...] * (1.0 - ab)
    o1 = jnp.maximum(s1_ref[...], 0.0) * ab + r1_ref[...] * (1.0 - ab)
    a0 = a0_ref[...]
    a1 = a1_ref[...]
    z0 = _leaky(jnp.dot(o0 * a0, G, preferred_element_type=jnp.float32))
    z1 = _leaky(jnp.dot(o1 * a0, G, preferred_element_type=jnp.float32))
    pe = jnp.dot(jax.nn.sigmoid(z0 - z1), E8, preferred_element_type=jnp.float32)
    c0_ref[...] = pe * o0 + (1.0 - pe) * o1
    y0 = _leaky(jnp.dot(o0 * a1, G, preferred_element_type=jnp.float32))
    y1 = _leaky(jnp.dot(o1 * a1, G, preferred_element_type=jnp.float32))
    qe = jnp.dot(jax.nn.sigmoid(y0 - y1), E8, preferred_element_type=jnp.float32)
    c1_ref[...] = qe * o0 + (1.0 - qe) * o1


def _run_post(out_all, r0, r1, ab, a0, a1, G, E8):
    BR = 1000
    grid = (N // BR,)
    blk = pl.BlockSpec
    out_shapes = (
        jax.ShapeDtypeStruct((N, 128), jnp.float32),
        jax.ShapeDtypeStruct((N, 128), jnp.float32),
    )
    in_specs = [
        blk((BR, 128), lambda i: (i, 0)),
        blk((BR, 128), lambda i: (i + N // BR, 0)),
        blk((BR, 128), lambda i: (i, 0)),
        blk((BR, 128), lambda i: (i, 0)),
        blk((1, 128), lambda i: (0, 0)),
        blk((1, 128), lambda i: (0, 0)),
        blk((1, 128), lambda i: (0, 0)),
        blk((128, 16), lambda i: (0, 0)),
        blk((16, 128), lambda i: (0, 0)),
    ]
    out_specs = (
        blk((BR, 128), lambda i: (i, 0)),
        blk((BR, 128), lambda i: (i, 0)),
    )
    return pl.pallas_call(
        _post_body, grid=grid, in_specs=in_specs, out_specs=out_specs,
        out_shape=out_shapes,
    )(out_all, out_all, r0, r1, ab, a0, a1, G, E8)


def kernel(x_r0, x_r1, rel_emb_r0, rel_emb_r1, W_node, W_rel_r0, W_rel_r1,
           attn_r0, attn_r1, res_W, res_b, res_alpha,
           prop_W_r0, prop_b_r0, prop_W_r1, prop_b_r1,
           edge_index_r0, edge_index_r1):
    f32 = jnp.float32
    Wr0 = W_rel_r0.reshape(64, H, 2, HID)
    Wr1 = W_rel_r1.reshape(64, H, 2, HID)
    Wd0 = Wr0[:, :, 0, :].reshape(64, 128)
    Ws0 = Wr0[:, :, 1, :].reshape(64, 128)
    Wd1 = Wr1[:, :, 0, :].reshape(64, 128)
    Ws1 = Wr1[:, :, 1, :].reshape(64, 128)
    jj = jnp.arange(128)[:, None]
    hh = jnp.arange(16)[None, :]
    G = (jj // HID == hh).astype(f32)
    E8 = G.T.copy()
    em0 = rel_emb_r0.reshape(1, 64)
    em1 = rel_emb_r1.reshape(1, 64)

    (f0, f1, ts0, td0, ts1, td1, r0, r1, p0, p1) = _run_pre(
        x_r0, x_r1, W_node, res_W, res_b.reshape(1, 128), em0, em1,
        Ws0, Wd0, Ws1, Wd1, prop_W_r0, prop_b_r0.reshape(1, 512),
        prop_W_r1, prop_b_r1.reshape(1, 512), G)

    src2 = jnp.stack([edge_index_r0[0], edge_index_r1[0]]).reshape(2 * NT, TE)
    dst2 = jnp.stack([edge_index_r0[1], edge_index_r1[1]]).reshape(2 * NT, TE)
    pad = TEP - TE
    srcp = jnp.pad(src2, ((0, 0), (0, pad))).reshape(2 * NT * NCH2, CB2)
    dstp = jnp.pad(dst2, ((0, 0), (0, pad)),
                   constant_values=N).reshape(2 * NT * NCH2, CB2)
    pidx = srcp | (dstp << 16)
    tsrc_all = jnp.pad(jnp.concatenate([ts0, ts1], axis=0), ((0, 16), (0, 0)))
    tdst_all = jnp.pad(jnp.concatenate([td0, td1], axis=0), ((0, 16), (0, 0)))
    feat_all = jnp.concatenate([f0, f1], axis=0)

    out_all = _run_sc(pidx, tsrc_all, tdst_all, feat_all)

    ab = jnp.broadcast_to(jax.nn.sigmoid(res_alpha), (1, 128)).astype(f32)
    c0, c1 = _run_post(out_all, r0, r1, ab,
                       attn_r0.reshape(1, 128), attn_r1.reshape(1, 128), G, E8)
    return (c0, c1, p0.reshape(512), p1.reshape(512))


# speedup vs baseline: 1.0251x; 1.0251x over previous
import functools

import jax
import jax.numpy as jnp
from jax import lax
from jax.experimental import pallas as pl
from jax.experimental.pallas import tpu as pltpu
from jax.experimental.pallas import tpu_sc as plsc

N = 10000
E = 320000
IN_DIM = 128
HID = 16
H = 8
NEG = 0.2

NT = 16
TE = E // NT
CB = 80
NCH = TE // CB
RB = 624
RW = 48


def _leaky(x):
    return jnp.where(x > 0, x, NEG * x)


def _vgather(v, idx):
    return lax.gather(
        v, idx[:, None],
        dimension_numbers=lax.GatherDimensionNumbers(
            offset_dims=(), collapsed_slice_dims=(0,), start_index_map=(0,)),
        slice_sizes=(1,), mode=lax.GatherScatterMode.PROMISE_IN_BOUNDS)


def _pre_body(x0_ref, x1_ref, Wn_ref, rW_ref, rb_ref, em0_ref, em1_ref,
              Ws0_ref, Wd0_ref, Ws1_ref, Wd1_ref, pW0_ref, pb0_ref,
              pW1_ref, pb1_ref, G_ref,
              f0_ref, f1_ref, ts0_ref, td0_ref, ts1_ref, td1_ref,
              r0_ref, r1_ref, p0_ref, p1_ref):
    Wn = Wn_ref[...]
    G = G_ref[...]
    x0 = x0_ref[...]
    x1 = x1_ref[...]
    f0 = jnp.dot(x0, Wn, preferred_element_type=jnp.float32)
    f1 = jnp.dot(x1, Wn, preferred_element_type=jnp.float32)
    f0_ref[...] = f0
    f1_ref[...] = f1
    rv_s0 = jnp.dot(em0_ref[...], Ws0_ref[...], preferred_element_type=jnp.float32)
    rv_d0 = jnp.dot(em0_ref[...], Wd0_ref[...], preferred_element_type=jnp.float32)
    rv_s1 = jnp.dot(em1_ref[...], Ws1_ref[...], preferred_element_type=jnp.float32)
    rv_d1 = jnp.dot(em1_ref[...], Wd1_ref[...], preferred_element_type=jnp.float32)
    ts0_ref[...] = jnp.dot(f0 * rv_s0, G, preferred_element_type=jnp.float32)
    td0_ref[...] = jnp.dot(f0 * rv_d0, G, preferred_element_type=jnp.float32)
    ts1_ref[...] = jnp.dot(f1 * rv_s1, G, preferred_element_type=jnp.float32)
    td1_ref[...] = jnp.dot(f1 * rv_d1, G, preferred_element_type=jnp.float32)
    r0_ref[...] = jnp.dot(x0, rW_ref[...], preferred_element_type=jnp.float32) + rb_ref[...]
    r1_ref[...] = jnp.dot(x1, rW_ref[...], preferred_element_type=jnp.float32) + rb_ref[...]
    p0_ref[...] = jnp.dot(em0_ref[...], pW0_ref[...], preferred_element_type=jnp.float32) + pb0_ref[...]
    p1_ref[...] = jnp.dot(em1_ref[...], pW1_ref[...], preferred_element_type=jnp.float32) + pb1_ref[...]


def _run_pre(x0, x1, Wn, rW, rb, em0, em1, Ws0, Wd0, Ws1, Wd1,
             pW0, pb0, pW1, pb1, G):
    BR = 1000
    grid = (N // BR,)
    row = lambda i: (i, 0)
    full = lambda i: (0, 0)
    blk = lambda shp, m: pl.BlockSpec(shp, m)
    out_shapes = (
        jax.ShapeDtypeStruct((N, 128), jnp.float32),
        jax.ShapeDtypeStruct((N, 128), jnp.float32),
        jax.ShapeDtypeStruct((N, 16), jnp.float32),
        jax.ShapeDtypeStruct((N, 16), jnp.float32),
        jax.ShapeDtypeStruct((N, 16), jnp.float32),
        jax.ShapeDtypeStruct((N, 16), jnp.float32),
        jax.ShapeDtypeStruct((N, 128), jnp.float32),
        jax.ShapeDtypeStruct((N, 128), jnp.float32),
        jax.ShapeDtypeStruct((1, 512), jnp.float32),
        jax.ShapeDtypeStruct((1, 512), jnp.float32),
    )
    in_specs = [
        blk((BR, 128), row), blk((BR, 128), row),
        blk((128, 128), full), blk((128, 128), full), blk((1, 128), full),
        blk((1, 64), full), blk((1, 64), full),
        blk((64, 128), full), blk((64, 128), full),
        blk((64, 128), full), blk((64, 128), full),
        blk((64, 512), full), blk((1, 512), full),
        blk((64, 512), full), blk((1, 512), full),
        blk((128, 16), full),
    ]
    out_specs = (
        blk((BR, 128), row), blk((BR, 128), row),
        blk((BR, 16), row), blk((BR, 16), row),
        blk((BR, 16), row), blk((BR, 16), row),
        blk((BR, 128), row), blk((BR, 128), row),
        blk((1, 512), full), blk((1, 512), full),
    )
    return pl.pallas_call(
        _pre_body, grid=grid, in_specs=in_specs, out_specs=out_specs,
        out_shape=out_shapes,
    )(x0, x1, Wn, rW, rb, em0, em1, Ws0, Wd0, Ws1, Wd1, pW0, pb0, pW1, pb1, G)


CB2 = 96
NCH2 = 210
TEP = NCH2 * CB2
NP = N + 16
NB2 = NCH2 // 2


def _sc_body(pidx_hbm, tsrc_hbm, tdst_hbm, feat_hbm, out_hbm,
             pidx0, pidx1, vdst0, vdst1, voffs0, voffs1, voffd0, voffd1,
             gs0, gs1, gd0, gd1, wv0, wv1, rr0, rr1, fb0, fb1,
             ssum_sh, out_sh,
             sa0, sb0, sc0, sd0, sa1, sb1, sc1, sd1):
    c = lax.axis_index("c")
    s = lax.axis_index("s")
    cN = c * N
    erow = (c * NT + s) * NCH2
    rbase = s * RB
    zero16 = jnp.zeros((16,), jnp.float32)
    hsplat = [jnp.full((16,), h, jnp.int32) for h in range(H)]
    pidx = (pidx0, pidx1)
    vdst = (vdst0, vdst1)
    voffs = (voffs0, voffs1)
    voffd = (voffd0, voffd1)
    gs = (gs0, gs1)
    gd = (gd0, gd1)
    wv = (wv0, wv1)
    rr = (rr0, rr1)
    fb = (fb0, fb1)
    sems = ((sa0, sb0, sc0, sd0), (sa1, sb1, sc1, sd1))

    def load_unpack(j, b):
        pltpu.sync_copy(pidx_hbm.at[erow + j], pidx[b])
        for i in range(CB2 // 16):
            p = pidx[b][pl.ds(i * 16, 16)]
            d = p >> 16
            sr = p & 0xFFFF
            vdst[b][pl.ds(i * 16, 16)] = d
            voffd[b][pl.ds(i * 16, 16)] = d + cN
            voffs[b][pl.ds(i * 16, 16)] = sr + cN

    def zf(i, _):
        fb0[i // 8, pl.ds((i % 8) * 16, 16)] = zero16
        return 0
    lax.fori_loop(0, CB2 * 8, zf, 0)

    def zw(i, _):
        wv0[i, :] = zero16
        return 0
    lax.fori_loop(0, CB2, zw, 0)

    def zrows(j, _):
        r = rbase + j * RW
        pltpu.sync_copy(wv0.at[pl.ds(0, RW)], ssum_sh.at[pl.ds(r, RW)])
        pltpu.sync_copy(fb0.at[pl.ds(0, RW)], out_sh.at[pl.ds(r, RW)])
        return 0
    lax.fori_loop(0, RB // RW, zrows, 0)

    @pl.when(s == NT - 1)
    def _():
        pltpu.sync_copy(wv0.at[pl.ds(0, 32)], ssum_sh.at[pl.ds(NT * RB, 32)])
        pltpu.sync_copy(fb0.at[pl.ds(0, 32)], out_sh.at[pl.ds(NT * RB, 32)])

    plsc.subcore_barrier()

    def a_compute_scatter(b):
        for e in range(CB2):
            v = gs[b][e, :] + gd[b][e, :]
            wv[b][e, :] = jnp.exp(jnp.where(v > 0, v, NEG * v))
        pltpu.sync_copy(wv[b], ssum_sh.at[vdst[b]], add=True)

    def passA(m, _):
        j0 = 2 * m
        load_unpack(j0, 0)
        d0 = (pltpu.async_copy(tsrc_hbm.at[voffs[0]], gs[0], sems[0][0]),
              pltpu.async_copy(tdst_hbm.at[voffd[0]], gd[0], sems[0][1]))
        load_unpack(j0 + 1, 1)
        d1 = (pltpu.async_copy(tsrc_hbm.at[voffs[1]], gs[1], sems[1][0]),
              pltpu.async_copy(tdst_hbm.at[voffd[1]], gd[1], sems[1][1]))
        d0[0].wait()
        d0[1].wait()
        a_compute_scatter(0)
        d1[0].wait()
        d1[1].wait()
        a_compute_scatter(1)
        return 0
    lax.fori_loop(0, NB2, passA, 0)

    plsc.subcore_barrier()

    def recip(j, _):
        r = rbase + j * RW
        pltpu.sync_copy(ssum_sh.at[pl.ds(r, RW)], wv0.at[pl.ds(0, RW)])
        for i in range(RW):
            wv0[i, :] = 1.0 / (wv0[i, :] + 1e-16)
        pltpu.sync_copy(wv0.at[pl.ds(0, RW)], ssum_sh.at[pl.ds(r, RW)])
        return 0
    lax.fori_loop(0, RB // RW, recip, 0)

    @pl.when(s == NT - 1)
    def _():
        pltpu.sync_copy(ssum_sh.at[pl.ds(NT * RB, 16)], wv0.at[pl.ds(0, 16)])
        for i in range(16):
            wv0[i, :] = 1.0 / (wv0[i, :] + 1e-16)
        pltpu.sync_copy(wv0.at[pl.ds(0, 16)], ssum_sh.at[pl.ds(NT * RB, 16)])

    plsc.subcore_barrier()

    def b_issue(b):
        return (pltpu.async_copy(feat_hbm.at[voffs[b]], fb[b], sems[b][0]),
                pltpu.async_copy(tsrc_hbm.at[voffs[b]], gs[b], sems[b][1]),
                pltpu.async_copy(tdst_hbm.at[voffd[b]], gd[b], sems[b][2]),
                pltpu.async_copy(ssum_sh.at[vdst[b]], rr[b], sems[b][3]))

    def b_compute_scatter(b):
        def scale(p, _):
            for q in range(2):
                e = 2 * p + q
                v = gs[b][e, :] + gd[b][e, :]
                w = jnp.exp(jnp.where(v > 0, v, NEG * v))
                av = w * rr[b][e, :]
                for h in range(H):
                    bv = _vgather(av, hsplat[h])
                    fb[b][e, pl.ds(h * 16, 16)] = fb[b][e, pl.ds(h * 16, 16)] * bv
            return 0
        lax.fori_loop(0, CB2 // 2, scale, 0)
        pltpu.sync_copy(fb[b], out_sh.at[vdst[b]], add=True)

    def passB(m, _):
        j0 = 2 * m
        load_unpack(j0, 0)
        d0 = b_issue(0)
        load_unpack(j0 + 1, 1)
        d1 = b_issue(1)
        for d in d0:
            d.wait()
        b_compute_scatter(0)
        for d in d1:
            d.wait()
        b_compute_scatter(1)
        return 0
    lax.fori_loop(0, NB2, passB, 0)

    plsc.subcore_barrier()

    def wout(j, _):
        r = rbase + j * RW
        pltpu.sync_copy(out_sh.at[pl.ds(r, RW)], fb0.at[pl.ds(0, RW)])
        pltpu.sync_copy(fb0.at[pl.ds(0, RW)], out_hbm.at[pl.ds(cN + r, RW)])
        return 0
    lax.fori_loop(0, RB // RW, wout, 0)

    @pl.when(s == NT - 1)
    def _():
        pltpu.sync_copy(out_sh.at[pl.ds(NT * RB, 16)], fb0.at[pl.ds(0, 16)])
        pltpu.sync_copy(fb0.at[pl.ds(0, 16)], out_hbm.at[pl.ds(cN + NT * RB, 16)])


def _run_sc(pidx, tsrc_all, tdst_all, feat_all):
    mesh = plsc.VectorSubcoreMesh(core_axis_name="c", subcore_axis_name="s",
                                  num_cores=2, num_subcores=NT)
    f = pl.kernel(
        _sc_body,
        out_type=jax.ShapeDtypeStruct((2 * N, 128), jnp.float32),
        mesh=mesh,
        compiler_params=pltpu.CompilerParams(use_tc_tiling_on_sc=False),
        scratch_types=(
            [pltpu.VMEM((CB2,), jnp.int32)] * 8
            + [pltpu.VMEM((CB2, 16), jnp.float32)] * 8
            + [pltpu.VMEM((CB2, 128), jnp.float32)] * 2
            + [pltpu.VMEM_SHARED((NP, 16), jnp.float32),
               pltpu.VMEM_SHARED((NP, 128), jnp.float32)]
            + [pltpu.SemaphoreType.DMA] * 8),
    )
    return f(pidx, tsrc_all, tdst_all, feat_all)


def _post_body(s0_ref, s1_ref, r0_ref, r1_ref, ab_ref, a0_ref, a1_ref, G_ref,
               E8_ref, c0_ref, c1_ref):
    ab = ab_ref[...]
    G = G_ref[...]
    E8 = E8_ref[...]
    o0 = jnp.maximum(s0_ref[...], 0.0) * ab + r0_ref[...] * (1.0 - ab)
    o1 = jnp.maximum(s1_ref[...], 0.0) * ab + r1_ref[...] * (1.0 - ab)
    a0 = a0_ref[...]
    a1 = a1_ref[...]
    z0 = _leaky(jnp.dot(o0 * a0, G, preferred_element_type=jnp.float32))
    z1 = _leaky(jnp.dot(o1 * a0, G, preferred_element_type=jnp.float32))
    pe = jnp.dot(jax.nn.sigmoid(z0 - z1), E8, preferred_element_type=jnp.float32)
    c0_ref[...] = pe * o0 + (1.0 - pe) * o1
    y0 = _leaky(jnp.dot(o0 * a1, G, preferred_element_type=jnp.float32))
    y1 = _leaky(jnp.dot(o1 * a1, G, preferred_element_type=jnp.float32))
    qe = jnp.dot(jax.nn.sigmoid(y0 - y1), E8, preferred_element_type=jnp.float32)
    c1_ref[...] = qe * o0 + (1.0 - qe) * o1


def _run_post(out_all, r0, r1, ab, a0, a1, G, E8):
    BR = 1000
    grid = (N // BR,)
    blk = pl.BlockSpec
    out_shapes = (
        jax.ShapeDtypeStruct((N, 128), jnp.float32),
        jax.ShapeDtypeStruct((N, 128), jnp.float32),
    )
    in_specs = [
        blk((BR, 128), lambda i: (i, 0)),
        blk((BR, 128), lambda i: (i + N // BR, 0)),
        blk((BR, 128), lambda i: (i, 0)),
        blk((BR, 128), lambda i: (i, 0)),
        blk((1, 128), lambda i: (0, 0)),
        blk((1, 128), lambda i: (0, 0)),
        blk((1, 128), lambda i: (0, 0)),
        blk((128, 16), lambda i: (0, 0)),
        blk((16, 128), lambda i: (0, 0)),
    ]
    out_specs = (
        blk((BR, 128), lambda i: (i, 0)),
        blk((BR, 128), lambda i: (i, 0)),
    )
    return pl.pallas_call(
        _post_body, grid=grid, in_specs=in_specs, out_specs=out_specs,
        out_shape=out_shapes,
    )(out_all, out_all, r0, r1, ab, a0, a1, G, E8)


def kernel(x_r0, x_r1, rel_emb_r0, rel_emb_r1, W_node, W_rel_r0, W_rel_r1,
           attn_r0, attn_r1, res_W, res_b, res_alpha,
           prop_W_r0, prop_b_r0, prop_W_r1, prop_b_r1,
           edge_index_r0, edge_index_r1):
    f32 = jnp.float32
    Wr0 = W_rel_r0.reshape(64, H, 2, HID)
    Wr1 = W_rel_r1.reshape(64, H, 2, HID)
    Wd0 = Wr0[:, :, 0, :].reshape(64, 128)
    Ws0 = Wr0[:, :, 1, :].reshape(64, 128)
    Wd1 = Wr1[:, :, 0, :].reshape(64, 128)
    Ws1 = Wr1[:, :, 1, :].reshape(64, 128)
    jj = jnp.arange(128)[:, None]
    hh = jnp.arange(16)[None, :]
    G = (jj // HID == hh).astype(f32)
    E8 = G.T.copy()
    em0 = rel_emb_r0.reshape(1, 64)
    em1 = rel_emb_r1.reshape(1, 64)

    (f0, f1, ts0, td0, ts1, td1, r0, r1, p0, p1) = _run_pre(
        x_r0, x_r1, W_node, res_W, res_b.reshape(1, 128), em0, em1,
        Ws0, Wd0, Ws1, Wd1, prop_W_r0, prop_b_r0.reshape(1, 512),
        prop_W_r1, prop_b_r1.reshape(1, 512), G)

    src2 = jnp.stack([edge_index_r0[0], edge_index_r1[0]]).reshape(2 * NT, TE)
    dst2 = jnp.stack([edge_index_r0[1], edge_index_r1[1]]).reshape(2 * NT, TE)
    pad = TEP - TE
    srcp = jnp.pad(src2, ((0, 0), (0, pad))).reshape(2 * NT * NCH2, CB2)
    dstp = jnp.pad(dst2, ((0, 0), (0, pad)),
                   constant_values=N).reshape(2 * NT * NCH2, CB2)
    pidx = srcp | (dstp << 16)
    tsrc_all = jnp.pad(jnp.concatenate([ts0, ts1], axis=0), ((0, 16), (0, 0)))
    tdst_all = jnp.pad(jnp.concatenate([td0, td1], axis=0), ((0, 16), (0, 0)))
    feat_all = jnp.concatenate([f0, f1], axis=0)

    out_all = _run_sc(pidx, tsrc_all, tdst_all, feat_all)

    ab = jnp.broadcast_to(jax.nn.sigmoid(res_alpha), (1, 128)).astype(f32)
    c0, c1 = _run_post(out_all, r0, r1, ab,
                       attn_r0.reshape(1, 128), attn_r1.reshape(1, 128), G, E8)
    return (c0, c1, p0.reshape(512), p1.reshape(512))
